# baseline (speedup 1.0000x reference)
import functools

import jax
import jax.numpy as jnp
from jax import lax
from jax.experimental import pallas as pl
from jax.experimental.pallas import tpu as pltpu
from jax.experimental.pallas import tpu_sc as plsc

N = 10000
E = 320000
D_IN = 128
DH = 32
D_OUT = 128

NC = 2
NS = 16
NW = NC * NS
L = 16

K = 80
NBLK = 125
EPW = K * NBLK
N_PAD = 10112
RPS = N_PAD // NS

_MESH = plsc.VectorSubcoreMesh(
    core_axis_name="c", subcore_axis_name="s", num_cores=NC, num_subcores=NS
)
_SC_PARAMS = pltpu.CompilerParams(
    needs_layout_passes=False, use_tc_tiling_on_sc=False,
    skip_device_barrier=True,
)


@functools.partial(
    pl.kernel,
    out_type=jax.ShapeDtypeStruct((NW, N_PAD), jnp.float32),
    mesh=_MESH,
    scratch_types=[
        pltpu.VMEM((N_PAD,), jnp.float32),
        pltpu.VMEM((N_PAD,), jnp.float32),
        pltpu.VMEM((NBLK, K), jnp.int32),
    ],
    compiler_params=_SC_PARAMS,
)
def _sc_degree(cols_ref, out_ref, hist, hist2, cbuf):
    c = lax.axis_index("c")
    s = lax.axis_index("s")
    wid = s * NC + c

    pltpu.sync_copy(cols_ref.at[wid], cbuf)

    zero16 = jnp.zeros((L,), jnp.float32)

    def zero_body(i, carry):
        for u in range(2):
            hist[pl.ds(i * 2 * L + u * L, L)] = zero16
            hist2[pl.ds(i * 2 * L + u * L, L)] = zero16
        return carry

    lax.fori_loop(0, N_PAD // (2 * L), zero_body, 0)

    ones16 = jnp.ones((L,), jnp.float32)
    hists = (hist, hist2)

    def edge_body(i, carry):
        for u in range(K // L):
            cv = cbuf[i, pl.ds(u * L, L)]
            plsc.addupdate_scatter(hists[u % 2], [cv], ones16)
        return carry

    lax.fori_loop(0, NBLK, edge_body, 0)

    def merge_body(i, carry):
        for u in range(2):
            sl = pl.ds(i * 2 * L + u * L, L)
            hist[sl] = hist[sl] + hist2[sl]
        return carry

    lax.fori_loop(0, N_PAD // (2 * L), merge_body, 0)

    pltpu.sync_copy(hist, out_ref.at[wid])


def _edge_pipeline(edge_ref, wid, rbuf, cbuf, gath0, gath1, sem0, sem1,
                   xin_sh, acc):
    pltpu.sync_copy(edge_ref.at[0, wid], rbuf)
    pltpu.sync_copy(edge_ref.at[1, wid], cbuf)

    gbufs = (gath0, gath1)
    sems = (sem0, sem1)

    def start_gather(i, b):
        pltpu.make_async_copy(xin_sh.at[rbuf.at[i]], gbufs[b], sems[b]).start()

    def finish_block(i, b):
        pltpu.make_async_copy(xin_sh.at[rbuf.at[i]], gbufs[b], sems[b]).wait()
        pltpu.sync_copy(gbufs[b], acc.at[cbuf.at[i]], add=True)

    start_gather(0, 0)

    def edge_body(j, carry):
        i0 = 2 * j
        start_gather(i0 + 1, 1)
        finish_block(i0, 0)
        start_gather(i0 + 2, 0)
        finish_block(i0 + 1, 1)
        return carry

    lax.fori_loop(0, (NBLK - 1) // 2, edge_body, 0)
    finish_block(NBLK - 1, 0)


@functools.partial(
    pl.kernel,
    out_type=jax.ShapeDtypeStruct((NC * N_PAD, DH), jnp.float32),
    mesh=_MESH,
    scratch_types=[
        pltpu.VMEM_SHARED((N_PAD, DH), jnp.float32),
        pltpu.VMEM_SHARED((N_PAD, DH), jnp.float32),
        pltpu.VMEM((NBLK, K), jnp.int32),
        pltpu.VMEM((NBLK, K), jnp.int32),
        pltpu.VMEM((K, DH), jnp.float32),
        pltpu.VMEM((K, DH), jnp.float32),
        pltpu.SemaphoreType.DMA,
        pltpu.SemaphoreType.DMA,
        pltpu.SemaphoreType.DMA,
    ],
    compiler_params=_SC_PARAMS,
)
def _sc_scatter1(xin_ref, edge_ref, zeros_ref, out_ref,
                 acc, xin_sh, rbuf, cbuf, gath0, gath1, sem0, sem1, sem2):
    c = lax.axis_index("c")
    s = lax.axis_index("s")
    wid = s * NC + c

    rsl = pl.ds(s * RPS, RPS)
    zcopy = pltpu.make_async_copy(zeros_ref.at[rsl], acc.at[rsl], sem1)
    zcopy.start()
    xcopy = pltpu.make_async_copy(xin_ref.at[rsl], xin_sh.at[rsl], sem2)
    xcopy.start()
    zcopy.wait()
    xcopy.wait()
    plsc.subcore_barrier()

    _edge_pipeline(edge_ref, wid, rbuf, cbuf, gath0, gath1, sem0, sem1,
                   xin_sh, acc)

    plsc.subcore_barrier()
    pltpu.sync_copy(acc.at[rsl], out_ref.at[pl.ds(c * N_PAD + s * RPS, RPS)])


@functools.partial(
    pl.kernel,
    out_type=jax.ShapeDtypeStruct((NC * N_PAD, DH), jnp.float32),
    mesh=_MESH,
    scratch_types=[
        pltpu.VMEM_SHARED((N_PAD, DH), jnp.float32),
        pltpu.VMEM_SHARED((N_PAD, DH), jnp.float32),
        pltpu.VMEM((NBLK, K), jnp.int32),
        pltpu.VMEM((NBLK, K), jnp.int32),
        pltpu.VMEM((K, DH), jnp.float32),
        pltpu.VMEM((K, DH), jnp.float32),
        pltpu.VMEM((RPS // 4, DH), jnp.float32),
        pltpu.VMEM((RPS // 4, DH), jnp.float32),
        pltpu.VMEM((RPS // 4, DH), jnp.float32),
        pltpu.VMEM((RPS // 4, DH), jnp.float32),
        pltpu.VMEM((RPS // 4, DH), jnp.float32),
        pltpu.VMEM((RPS // 4, DH), jnp.float32),
        pltpu.VMEM((RPS // 4, DH), jnp.float32),
        pltpu.VMEM((RPS // 4, DH), jnp.float32),
        pltpu.VMEM((DH,), jnp.float32),
        pltpu.SemaphoreType.DMA,
        pltpu.SemaphoreType.DMA,
        pltpu.SemaphoreType.DMA,
        pltpu.SemaphoreType.DMA,
        pltpu.SemaphoreType.DMA,
        pltpu.SemaphoreType.DMA,
        pltpu.SemaphoreType.DMA,
        pltpu.SemaphoreType.DMA,
        pltpu.SemaphoreType.DMA,
        pltpu.SemaphoreType.DMA,
    ],
    compiler_params=_SC_PARAMS,
)
def _sc_scatter2(p_ref, h0s_ref, disx_ref, b1_ref, edge_ref, zeros_ref,
                 out_ref, acc, xin_sh, rbuf, cbuf, gath0, gath1,
                 paA, pbA, h0bA, dxbA, paB, pbB, h0bB, dxbB, bbuf,
                 semA0, semA1, semA2, semA3, semB0, semB1, semB2, semB3,
                 semZ, semW):
    c = lax.axis_index("c")
    s = lax.axis_index("s")
    wid = s * NC + c

    HC = RPS // 4
    sets = ((paA, pbA, h0bA, dxbA), (paB, pbB, h0bB, dxbB))
    ssems = ((semA0, semA1, semA2, semA3), (semB0, semB1, semB2, semB3))
    rsl = pl.ds(s * RPS, RPS)
    zcopy = pltpu.make_async_copy(zeros_ref.at[rsl], acc.at[rsl], semZ)
    zcopy.start()
    pltpu.sync_copy(b1_ref, bbuf)

    def _load_descs(ch, st):
        off = pl.ds(s * RPS + ch * HC, HC)
        pa, pb, h0b, dxb = sets[st]
        s0, s1, s2, s3 = ssems[st]
        return (
            pltpu.make_async_copy(p_ref.at[pl.ds(s * RPS + ch * HC, HC)], pa,
                                  s0),
            pltpu.make_async_copy(
                p_ref.at[pl.ds(N_PAD + s * RPS + ch * HC, HC)], pb, s1),
            pltpu.make_async_copy(h0s_ref.at[off], h0b, s2),
            pltpu.make_async_copy(disx_ref.at[off], dxb, s3),
        )

    def _wdesc(ch, st):
        off = pl.ds(s * RPS + ch * HC, HC)
        return pltpu.make_async_copy(sets[st][0], xin_sh.at[off], semW)

    def _mk_hs_body(st):
        pa, pb, h0b, dxb = sets[st]

        def hs_body(i2, carry):
            for r in range(2):
                i = 2 * i2 + r
                for u in range(DH // L):
                    csl = pl.ds(u * L, L)
                    d = dxb[i, csl]
                    v = d * (pa[i, csl] + pb[i, csl] + h0b[i, csl]) + bbuf[csl]
                    pa[i, csl] = d * jnp.maximum(v, 0.0)
            return carry

        return hs_body

    for d in _load_descs(0, 0):
        d.start()
    for ch in range(4):
        st = ch % 2
        if ch + 1 < 4:
            if ch >= 1:
                _wdesc(ch - 1, 1 - st).wait()
            for d in _load_descs(ch + 1, 1 - st):
                d.start()
        for d in _load_descs(ch, st):
            d.wait()
        lax.fori_loop(0, HC // 2, _mk_hs_body(st), 0)
        _wdesc(ch, st).start()
    _wdesc(2, 0).wait()
    _wdesc(3, 1).wait()

    zcopy.wait()
    plsc.subcore_barrier()

    _edge_pipeline(edge_ref, wid, rbuf, cbuf, gath0, gath1, semA0, semB0,
                   xin_sh, acc)

    plsc.subcore_barrier()

    def _eload_descs(ch, st):
        off = pl.ds(s * RPS + ch * HC, HC)
        pa, pb, h0b, dxb = sets[st]
        s0, s1, s2, s3 = ssems[st]
        return (
            pltpu.make_async_copy(acc.at[off], pb, s0),
            pltpu.make_async_copy(disx_ref.at[off], dxb, s1),
            pltpu.make_async_copy(xin_sh.at[off], h0b, s2),
        )

    def _ewdesc(ch, st):
        off = pl.ds(c * N_PAD + s * RPS + ch * HC, HC)
        return pltpu.make_async_copy(sets[st][1], out_ref.at[off], semW)

    def _mk_ep_body(st, with_hs):
        pa, pb, h0b, dxb = sets[st]

        def ep_body(i2, carry):
            for r in range(2):
                i = 2 * i2 + r
                for u in range(DH // L):
                    csl = pl.ds(u * L, L)
                    q = pb[i, csl]
                    if with_hs:
                        q = q + h0b[i, csl]
                    pb[i, csl] = dxb[i, csl] * q
            return carry

        return ep_body

    for d in _eload_descs(0, 0):
        d.start()
    for ch in range(4):
        st = ch % 2
        if ch + 1 < 4:
            if ch >= 1:
                _ewdesc(ch - 1, 1 - st).wait()
            for d in _eload_descs(ch + 1, 1 - st):
                d.start()
        for d in _eload_descs(ch, st):
            d.wait()

        @pl.when(c == 0)
        def _():
            lax.fori_loop(0, HC // 2, _mk_ep_body(st, True), 0)

        @pl.when(c != 0)
        def _():
            lax.fori_loop(0, HC // 2, _mk_ep_body(st, False), 0)

        _ewdesc(ch, st).start()
    _ewdesc(2, 0).wait()
    _ewdesc(3, 1).wait()


def _tc_mm_body(x_ref, w1_ref, h0_ref):
    h0_ref[...] = jnp.dot(x_ref[...], w1_ref[...],
                          preferred_element_type=jnp.float32)


def _tc_scale_body(parts_ref, h0_ref, h0s_ref, disx_ref):
    deg = jnp.sum(parts_ref[...], axis=0) + 1.0
    disx = jnp.broadcast_to(lax.rsqrt(deg)[:, None], (N_PAD, DH))
    h0p = jnp.concatenate(
        [h0_ref[...], jnp.zeros((N_PAD - N, DH), jnp.float32)], axis=0)
    h0s_ref[...] = disx * h0p
    disx_ref[...] = disx


def _tc_final_body(r_ref, w2_ref, b2_ref, out_ref):
    rall = r_ref[...]
    agg = rall[0:N] + rall[N_PAD:N_PAD + N]
    out = jnp.dot(agg, w2_ref[...], preferred_element_type=jnp.float32)
    out_ref[...] = out + b2_ref[...]


_tc_mm = pl.pallas_call(
    _tc_mm_body,
    out_shape=jax.ShapeDtypeStruct((N, DH), jnp.float32),
)

_tc_scale = pl.pallas_call(
    _tc_scale_body,
    out_shape=(
        jax.ShapeDtypeStruct((N_PAD, DH), jnp.float32),
        jax.ShapeDtypeStruct((N_PAD, DH), jnp.float32),
    ),
)

_tc_final = pl.pallas_call(
    _tc_final_body,
    out_shape=jax.ShapeDtypeStruct((N, D_OUT), jnp.float32),
)


def kernel(x, edge_index, W1, b1, W2, b2):
    edge3 = edge_index.reshape(2, NW, NBLK, K)
    cols3 = edge_index[1].reshape(NW, NBLK, K)
    zeros2 = jnp.zeros((N_PAD, DH), jnp.float32)

    parts = _sc_degree(cols3)
    h0 = _tc_mm(x, W1)
    h0s, disx = _tc_scale(parts, h0)
    p = _sc_scatter1(h0s, edge3, zeros2)
    r = _sc_scatter2(p, h0s, disx, b1, edge3, zeros2)
    out = _tc_final(r, W2, b2.reshape(1, D_OUT))
    return out


# speedup vs baseline: 1.0747x; 1.0747x over previous
import functools

import jax
import jax.numpy as jnp
from jax import lax
from jax.experimental import pallas as pl
from jax.experimental.pallas import tpu as pltpu
from jax.experimental.pallas import tpu_sc as plsc

N = 10000
E = 320000
D_IN = 128
DH = 32
D_OUT = 128

NC = 2
NS = 16
NW = NC * NS
L = 16

K = 80
NBLK = 125
EPW = K * NBLK
N_PAD = 10112
RPS = N_PAD // NS

_MESH = plsc.VectorSubcoreMesh(
    core_axis_name="c", subcore_axis_name="s", num_cores=NC, num_subcores=NS
)
_SC_PARAMS = pltpu.CompilerParams(
    needs_layout_passes=False, use_tc_tiling_on_sc=False,
    skip_device_barrier=True,
)


@functools.partial(
    pl.kernel,
    out_type=jax.ShapeDtypeStruct((NW, N_PAD), jnp.float32),
    mesh=_MESH,
    scratch_types=[
        pltpu.VMEM((N_PAD,), jnp.float32),
        pltpu.VMEM((N_PAD,), jnp.float32),
        pltpu.VMEM((NBLK, K), jnp.int32),
    ],
    compiler_params=_SC_PARAMS,
)
def _sc_degree(edge_ref, out_ref, hist, hist2, cbuf):
    c = lax.axis_index("c")
    s = lax.axis_index("s")
    wid = s * NC + c

    pltpu.sync_copy(edge_ref.at[1, wid], cbuf)

    zero16 = jnp.zeros((L,), jnp.float32)

    def zero_body(i, carry):
        for u in range(2):
            hist[pl.ds(i * 2 * L + u * L, L)] = zero16
            hist2[pl.ds(i * 2 * L + u * L, L)] = zero16
        return carry

    lax.fori_loop(0, N_PAD // (2 * L), zero_body, 0)

    ones16 = jnp.ones((L,), jnp.float32)
    hists = (hist, hist2)

    def edge_body(i, carry):
        for u in range(K // L):
            cv = cbuf[i, pl.ds(u * L, L)]
            plsc.addupdate_scatter(hists[u % 2], [cv], ones16)
        return carry

    lax.fori_loop(0, NBLK, edge_body, 0)

    def merge_body(i, carry):
        for u in range(2):
            sl = pl.ds(i * 2 * L + u * L, L)
            hist[sl] = hist[sl] + hist2[sl]
        return carry

    lax.fori_loop(0, N_PAD // (2 * L), merge_body, 0)

    pltpu.sync_copy(hist, out_ref.at[wid])


def _edge_pipeline(edge_ref, wid, rbuf, cbuf, gath0, gath1, sem0, sem1,
                   xin_sh, acc):
    pltpu.sync_copy(edge_ref.at[0, wid], rbuf)
    pltpu.sync_copy(edge_ref.at[1, wid], cbuf)

    gbufs = (gath0, gath1)
    sems = (sem0, sem1)

    def start_gather(i, b):
        pltpu.make_async_copy(xin_sh.at[rbuf.at[i]], gbufs[b], sems[b]).start()

    def finish_block(i, b):
        pltpu.make_async_copy(xin_sh.at[rbuf.at[i]], gbufs[b], sems[b]).wait()
        pltpu.sync_copy(gbufs[b], acc.at[cbuf.at[i]], add=True)

    start_gather(0, 0)

    def edge_body(j, carry):
        i0 = 2 * j
        start_gather(i0 + 1, 1)
        finish_block(i0, 0)
        start_gather(i0 + 2, 0)
        finish_block(i0 + 1, 1)
        return carry

    lax.fori_loop(0, (NBLK - 1) // 2, edge_body, 0)
    finish_block(NBLK - 1, 0)


@functools.partial(
    pl.kernel,
    out_type=jax.ShapeDtypeStruct((NC * N_PAD, DH), jnp.float32),
    mesh=_MESH,
    scratch_types=[
        pltpu.VMEM_SHARED((N_PAD, DH), jnp.float32),
        pltpu.VMEM_SHARED((N_PAD, DH), jnp.float32),
        pltpu.VMEM((NBLK, K), jnp.int32),
        pltpu.VMEM((NBLK, K), jnp.int32),
        pltpu.VMEM((K, DH), jnp.float32),
        pltpu.VMEM((K, DH), jnp.float32),
        pltpu.SemaphoreType.DMA,
        pltpu.SemaphoreType.DMA,
        pltpu.SemaphoreType.DMA,
    ],
    compiler_params=_SC_PARAMS,
)
def _sc_scatter1(xin_ref, edge_ref, zeros_ref, out_ref,
                 acc, xin_sh, rbuf, cbuf, gath0, gath1, sem0, sem1, sem2):
    c = lax.axis_index("c")
    s = lax.axis_index("s")
    wid = s * NC + c

    rsl = pl.ds(s * RPS, RPS)
    zcopy = pltpu.make_async_copy(zeros_ref.at[rsl], acc.at[rsl], sem1)
    zcopy.start()
    xcopy = pltpu.make_async_copy(xin_ref.at[rsl], xin_sh.at[rsl], sem2)
    xcopy.start()
    zcopy.wait()
    xcopy.wait()
    plsc.subcore_barrier()

    _edge_pipeline(edge_ref, wid, rbuf, cbuf, gath0, gath1, sem0, sem1,
                   xin_sh, acc)

    plsc.subcore_barrier()
    pltpu.sync_copy(acc.at[rsl], out_ref.at[pl.ds(c * N_PAD + s * RPS, RPS)])


@functools.partial(
    pl.kernel,
    out_type=jax.ShapeDtypeStruct((NC * N_PAD, DH), jnp.float32),
    mesh=_MESH,
    scratch_types=[
        pltpu.VMEM_SHARED((N_PAD, DH), jnp.float32),
        pltpu.VMEM_SHARED((N_PAD, DH), jnp.float32),
        pltpu.VMEM((NBLK, K), jnp.int32),
        pltpu.VMEM((NBLK, K), jnp.int32),
        pltpu.VMEM((K, DH), jnp.float32),
        pltpu.VMEM((K, DH), jnp.float32),
        pltpu.VMEM((RPS // 4, DH), jnp.float32),
        pltpu.VMEM((RPS // 4, DH), jnp.float32),
        pltpu.VMEM((RPS // 4, DH), jnp.float32),
        pltpu.VMEM((RPS // 4, DH), jnp.float32),
        pltpu.VMEM((RPS // 4, DH), jnp.float32),
        pltpu.VMEM((RPS // 4, DH), jnp.float32),
        pltpu.VMEM((RPS // 4, DH), jnp.float32),
        pltpu.VMEM((RPS // 4, DH), jnp.float32),
        pltpu.VMEM((DH,), jnp.float32),
        pltpu.SemaphoreType.DMA,
        pltpu.SemaphoreType.DMA,
        pltpu.SemaphoreType.DMA,
        pltpu.SemaphoreType.DMA,
        pltpu.SemaphoreType.DMA,
        pltpu.SemaphoreType.DMA,
        pltpu.SemaphoreType.DMA,
        pltpu.SemaphoreType.DMA,
        pltpu.SemaphoreType.DMA,
        pltpu.SemaphoreType.DMA,
    ],
    compiler_params=_SC_PARAMS,
)
def _sc_scatter2(p_ref, h0s_ref, disx_ref, b1_ref, edge_ref, zeros_ref,
                 out_ref, acc, xin_sh, rbuf, cbuf, gath0, gath1,
                 paA, pbA, h0bA, dxbA, paB, pbB, h0bB, dxbB, bbuf,
                 semA0, semA1, semA2, semA3, semB0, semB1, semB2, semB3,
                 semZ, semW):
    c = lax.axis_index("c")
    s = lax.axis_index("s")
    wid = s * NC + c

    HC = RPS // 4
    sets = ((paA, pbA, h0bA, dxbA), (paB, pbB, h0bB, dxbB))
    ssems = ((semA0, semA1, semA2, semA3), (semB0, semB1, semB2, semB3))
    rsl = pl.ds(s * RPS, RPS)
    zcopy = pltpu.make_async_copy(zeros_ref.at[rsl], acc.at[rsl], semZ)
    zcopy.start()
    pltpu.sync_copy(b1_ref, bbuf)

    def _load_descs(ch, st):
        off = pl.ds(s * RPS + ch * HC, HC)
        pa, pb, h0b, dxb = sets[st]
        s0, s1, s2, s3 = ssems[st]
        return (
            pltpu.make_async_copy(p_ref.at[pl.ds(s * RPS + ch * HC, HC)], pa,
                                  s0),
            pltpu.make_async_copy(
                p_ref.at[pl.ds(N_PAD + s * RPS + ch * HC, HC)], pb, s1),
            pltpu.make_async_copy(h0s_ref.at[off], h0b, s2),
            pltpu.make_async_copy(disx_ref.at[off], dxb, s3),
        )

    def _wdesc(ch, st):
        off = pl.ds(s * RPS + ch * HC, HC)
        return pltpu.make_async_copy(sets[st][0], xin_sh.at[off], semW)

    def _mk_hs_body(st):
        pa, pb, h0b, dxb = sets[st]

        def hs_body(i2, carry):
            for r in range(2):
                i = 2 * i2 + r
                for u in range(DH // L):
                    csl = pl.ds(u * L, L)
                    d = dxb[i, csl]
                    v = d * (pa[i, csl] + pb[i, csl] + h0b[i, csl]) + bbuf[csl]
                    pa[i, csl] = d * jnp.maximum(v, 0.0)
            return carry

        return hs_body

    for d in _load_descs(0, 0):
        d.start()
    for ch in range(4):
        st = ch % 2
        if ch + 1 < 4:
            if ch >= 1:
                _wdesc(ch - 1, 1 - st).wait()
            for d in _load_descs(ch + 1, 1 - st):
                d.start()
        for d in _load_descs(ch, st):
            d.wait()
        lax.fori_loop(0, HC // 2, _mk_hs_body(st), 0)
        _wdesc(ch, st).start()
    _wdesc(2, 0).wait()
    _wdesc(3, 1).wait()

    zcopy.wait()
    plsc.subcore_barrier()

    _edge_pipeline(edge_ref, wid, rbuf, cbuf, gath0, gath1, semA0, semB0,
                   xin_sh, acc)

    plsc.subcore_barrier()

    def _eload_descs(ch, st):
        off = pl.ds(s * RPS + ch * HC, HC)
        pa, pb, h0b, dxb = sets[st]
        s0, s1, s2, s3 = ssems[st]
        return (
            pltpu.make_async_copy(acc.at[off], pb, s0),
            pltpu.make_async_copy(disx_ref.at[off], dxb, s1),
            pltpu.make_async_copy(xin_sh.at[off], h0b, s2),
        )

    def _ewdesc(ch, st):
        off = pl.ds(c * N_PAD + s * RPS + ch * HC, HC)
        return pltpu.make_async_copy(sets[st][1], out_ref.at[off], semW)

    def _mk_ep_body(st, with_hs):
        pa, pb, h0b, dxb = sets[st]

        def ep_body(i2, carry):
            for r in range(2):
                i = 2 * i2 + r
                for u in range(DH // L):
                    csl = pl.ds(u * L, L)
                    q = pb[i, csl]
                    if with_hs:
                        q = q + h0b[i, csl]
                    pb[i, csl] = dxb[i, csl] * q
            return carry

        return ep_body

    for d in _eload_descs(0, 0):
        d.start()
    for ch in range(4):
        st = ch % 2
        if ch + 1 < 4:
            if ch >= 1:
                _ewdesc(ch - 1, 1 - st).wait()
            for d in _eload_descs(ch + 1, 1 - st):
                d.start()
        for d in _eload_descs(ch, st):
            d.wait()

        @pl.when(c == 0)
        def _():
            lax.fori_loop(0, HC // 2, _mk_ep_body(st, True), 0)

        @pl.when(c != 0)
        def _():
            lax.fori_loop(0, HC // 2, _mk_ep_body(st, False), 0)

        _ewdesc(ch, st).start()
    _ewdesc(2, 0).wait()
    _ewdesc(3, 1).wait()


def _tc_mm_body(x_ref, w1_ref, h0_ref):
    h0_ref[...] = jnp.dot(x_ref[...], w1_ref[...],
                          preferred_element_type=jnp.float32)


def _tc_scale_body(parts_ref, h0_ref, h0s_ref, disx_ref):
    deg = jnp.sum(parts_ref[...], axis=0) + 1.0
    disx = jnp.broadcast_to(lax.rsqrt(deg)[:, None], (N_PAD, DH))
    h0p = jnp.concatenate(
        [h0_ref[...], jnp.zeros((N_PAD - N, DH), jnp.float32)], axis=0)
    h0s_ref[...] = disx * h0p
    disx_ref[...] = disx


def _tc_final_body(r_ref, w2_ref, b2_ref, out_ref):
    rall = r_ref[...]
    agg = rall[0:N] + rall[N_PAD:N_PAD + N]
    out = jnp.dot(agg, w2_ref[...], preferred_element_type=jnp.float32)
    out_ref[...] = out + b2_ref[...]


_tc_mm = pl.pallas_call(
    _tc_mm_body,
    out_shape=jax.ShapeDtypeStruct((N, DH), jnp.float32),
)

_tc_scale = pl.pallas_call(
    _tc_scale_body,
    out_shape=(
        jax.ShapeDtypeStruct((N_PAD, DH), jnp.float32),
        jax.ShapeDtypeStruct((N_PAD, DH), jnp.float32),
    ),
)

_tc_final = pl.pallas_call(
    _tc_final_body,
    out_shape=jax.ShapeDtypeStruct((N, D_OUT), jnp.float32),
)


def kernel(x, edge_index, W1, b1, W2, b2):
    edge3 = edge_index.reshape(2, NW, NBLK, K)
    zeros2 = jnp.zeros((N_PAD, DH), jnp.float32)

    parts = _sc_degree(edge3)
    h0 = _tc_mm(x, W1)
    h0s, disx = _tc_scale(parts, h0)
    p = _sc_scatter1(h0s, edge3, zeros2)
    r = _sc_scatter2(p, h0s, disx, b1, edge3, zeros2)
    out = _tc_final(r, W2, b2.reshape(1, D_OUT))
    return out


# speedup vs baseline: 1.0899x; 1.0142x over previous
import functools

import jax
import jax.numpy as jnp
from jax import lax
from jax.experimental import pallas as pl
from jax.experimental.pallas import tpu as pltpu
from jax.experimental.pallas import tpu_sc as plsc

N = 10000
E = 320000
D_IN = 128
DH = 32
D_OUT = 128

NC = 2
NS = 16
NW = NC * NS
L = 16

K = 80
NBLK = 125
EPW = K * NBLK
N_PAD = 10112
RPS = N_PAD // NS

_MESH = plsc.VectorSubcoreMesh(
    core_axis_name="c", subcore_axis_name="s", num_cores=NC, num_subcores=NS
)
_SC_PARAMS = pltpu.CompilerParams(
    needs_layout_passes=False, use_tc_tiling_on_sc=False,
    skip_device_barrier=True,
)


@functools.partial(
    pl.kernel,
    out_type=jax.ShapeDtypeStruct((NW, N_PAD), jnp.float32),
    mesh=_MESH,
    scratch_types=[
        pltpu.VMEM((N_PAD,), jnp.float32),
        pltpu.VMEM((NBLK, K), jnp.int32),
    ],
    compiler_params=_SC_PARAMS,
)
def _sc_degree(edge_ref, out_ref, hist, cbuf):
    c = lax.axis_index("c")
    s = lax.axis_index("s")
    wid = s * NC + c

    pltpu.sync_copy(edge_ref.at[1, wid], cbuf)

    zero16 = jnp.zeros((L,), jnp.float32)

    def zero_body(i, carry):
        for u in range(4):
            hist[pl.ds(i * 4 * L + u * L, L)] = zero16
        return carry

    lax.fori_loop(0, N_PAD // (4 * L), zero_body, 0)

    ones16 = jnp.ones((L,), jnp.float32)

    def edge_body(i, carry):
        for u in range(K // L):
            cv = cbuf[i, pl.ds(u * L, L)]
            plsc.addupdate_scatter(hist, [cv], ones16)
        return carry

    lax.fori_loop(0, NBLK, edge_body, 0)

    pltpu.sync_copy(hist, out_ref.at[wid])


def _edge_pipeline(edge_ref, wid, rbuf, cbuf, gath0, gath1, sem0, sem1,
                   xin_sh, acc):
    pltpu.sync_copy(edge_ref.at[0, wid], rbuf)
    pltpu.sync_copy(edge_ref.at[1, wid], cbuf)

    gbufs = (gath0, gath1)
    sems = (sem0, sem1)

    def start_gather(i, b):
        pltpu.make_async_copy(xin_sh.at[rbuf.at[i]], gbufs[b], sems[b]).start()

    def finish_block(i, b):
        pltpu.make_async_copy(xin_sh.at[rbuf.at[i]], gbufs[b], sems[b]).wait()
        pltpu.sync_copy(gbufs[b], acc.at[cbuf.at[i]], add=True)

    start_gather(0, 0)

    def edge_body(j, carry):
        i0 = 2 * j
        start_gather(i0 + 1, 1)
        finish_block(i0, 0)
        start_gather(i0 + 2, 0)
        finish_block(i0 + 1, 1)
        return carry

    lax.fori_loop(0, (NBLK - 1) // 2, edge_body, 0)
    finish_block(NBLK - 1, 0)


@functools.partial(
    pl.kernel,
    out_type=jax.ShapeDtypeStruct((NC * N_PAD, DH), jnp.float32),
    mesh=_MESH,
    scratch_types=[
        pltpu.VMEM_SHARED((N_PAD, DH), jnp.float32),
        pltpu.VMEM_SHARED((N_PAD, DH), jnp.float32),
        pltpu.VMEM((NBLK, K), jnp.int32),
        pltpu.VMEM((NBLK, K), jnp.int32),
        pltpu.VMEM((K, DH), jnp.float32),
        pltpu.VMEM((K, DH), jnp.float32),
        pltpu.SemaphoreType.DMA,
        pltpu.SemaphoreType.DMA,
        pltpu.SemaphoreType.DMA,
    ],
    compiler_params=_SC_PARAMS,
)
def _sc_scatter1(xin_ref, edge_ref, zeros_ref, out_ref,
                 acc, xin_sh, rbuf, cbuf, gath0, gath1, sem0, sem1, sem2):
    c = lax.axis_index("c")
    s = lax.axis_index("s")
    wid = s * NC + c

    rsl = pl.ds(s * RPS, RPS)
    zcopy = pltpu.make_async_copy(zeros_ref.at[rsl], acc.at[rsl], sem1)
    zcopy.start()
    xcopy = pltpu.make_async_copy(xin_ref.at[rsl], xin_sh.at[rsl], sem2)
    xcopy.start()
    zcopy.wait()
    xcopy.wait()
    plsc.subcore_barrier()

    _edge_pipeline(edge_ref, wid, rbuf, cbuf, gath0, gath1, sem0, sem1,
                   xin_sh, acc)

    plsc.subcore_barrier()
    pltpu.sync_copy(acc.at[rsl], out_ref.at[pl.ds(c * N_PAD + s * RPS, RPS)])


@functools.partial(
    pl.kernel,
    out_type=jax.ShapeDtypeStruct((NC * N_PAD, DH), jnp.float32),
    mesh=_MESH,
    scratch_types=[
        pltpu.VMEM_SHARED((N_PAD, DH), jnp.float32),
        pltpu.VMEM_SHARED((N_PAD, DH), jnp.float32),
        pltpu.VMEM((NBLK, K), jnp.int32),
        pltpu.VMEM((NBLK, K), jnp.int32),
        pltpu.VMEM((K, DH), jnp.float32),
        pltpu.VMEM((K, DH), jnp.float32),
        pltpu.VMEM((RPS // 4, DH), jnp.float32),
        pltpu.VMEM((RPS // 4, DH), jnp.float32),
        pltpu.VMEM((RPS // 4, DH), jnp.float32),
        pltpu.VMEM((RPS // 4, DH), jnp.float32),
        pltpu.VMEM((RPS // 4, DH), jnp.float32),
        pltpu.VMEM((RPS // 4, DH), jnp.float32),
        pltpu.VMEM((RPS // 4, DH), jnp.float32),
        pltpu.VMEM((RPS // 4, DH), jnp.float32),
        pltpu.VMEM((DH,), jnp.float32),
        pltpu.SemaphoreType.DMA,
        pltpu.SemaphoreType.DMA,
        pltpu.SemaphoreType.DMA,
        pltpu.SemaphoreType.DMA,
        pltpu.SemaphoreType.DMA,
        pltpu.SemaphoreType.DMA,
        pltpu.SemaphoreType.DMA,
        pltpu.SemaphoreType.DMA,
        pltpu.SemaphoreType.DMA,
        pltpu.SemaphoreType.DMA,
    ],
    compiler_params=_SC_PARAMS,
)
def _sc_scatter2(p_ref, h0s_ref, disx_ref, b1_ref, edge_ref, zeros_ref,
                 out_ref, acc, xin_sh, rbuf, cbuf, gath0, gath1,
                 paA, pbA, h0bA, dxbA, paB, pbB, h0bB, dxbB, bbuf,
                 semA0, semA1, semA2, semA3, semB0, semB1, semB2, semB3,
                 semZ, semW):
    c = lax.axis_index("c")
    s = lax.axis_index("s")
    wid = s * NC + c

    HC = RPS // 4
    sets = ((paA, pbA, h0bA, dxbA), (paB, pbB, h0bB, dxbB))
    ssems = ((semA0, semA1, semA2, semA3), (semB0, semB1, semB2, semB3))
    rsl = pl.ds(s * RPS, RPS)
    zcopy = pltpu.make_async_copy(zeros_ref.at[rsl], acc.at[rsl], semZ)
    zcopy.start()
    pltpu.sync_copy(b1_ref, bbuf)

    def _load_descs(ch, st):
        off = pl.ds(s * RPS + ch * HC, HC)
        pa, pb, h0b, dxb = sets[st]
        s0, s1, s2, s3 = ssems[st]
        return (
            pltpu.make_async_copy(p_ref.at[pl.ds(s * RPS + ch * HC, HC)], pa,
                                  s0),
            pltpu.make_async_copy(
                p_ref.at[pl.ds(N_PAD + s * RPS + ch * HC, HC)], pb, s1),
            pltpu.make_async_copy(h0s_ref.at[off], h0b, s2),
            pltpu.make_async_copy(disx_ref.at[off], dxb, s3),
        )

    def _wdesc(ch, st):
        off = pl.ds(s * RPS + ch * HC, HC)
        return pltpu.make_async_copy(sets[st][0], xin_sh.at[off], semW)

    def _mk_hs_body(st):
        pa, pb, h0b, dxb = sets[st]

        def hs_body(i2, carry):
            for r in range(2):
                i = 2 * i2 + r
                for u in range(DH // L):
                    csl = pl.ds(u * L, L)
                    d = dxb[i, csl]
                    v = d * (pa[i, csl] + pb[i, csl] + h0b[i, csl]) + bbuf[csl]
                    pa[i, csl] = d * jnp.maximum(v, 0.0)
            return carry

        return hs_body

    for d in _load_descs(0, 0):
        d.start()
    for ch in range(4):
        st = ch % 2
        if ch + 1 < 4:
            if ch >= 1:
                _wdesc(ch - 1, 1 - st).wait()
            for d in _load_descs(ch + 1, 1 - st):
                d.start()
        for d in _load_descs(ch, st):
            d.wait()
        lax.fori_loop(0, HC // 2, _mk_hs_body(st), 0)
        _wdesc(ch, st).start()
    _wdesc(2, 0).wait()
    _wdesc(3, 1).wait()

    zcopy.wait()
    plsc.subcore_barrier()

    _edge_pipeline(edge_ref, wid, rbuf, cbuf, gath0, gath1, semA0, semB0,
                   xin_sh, acc)

    plsc.subcore_barrier()

    def _eload_descs(ch, st):
        off = pl.ds(s * RPS + ch * HC, HC)
        pa, pb, h0b, dxb = sets[st]
        s0, s1, s2, s3 = ssems[st]
        return (
            pltpu.make_async_copy(acc.at[off], pb, s0),
            pltpu.make_async_copy(disx_ref.at[off], dxb, s1),
            pltpu.make_async_copy(xin_sh.at[off], h0b, s2),
        )

    def _ewdesc(ch, st):
        off = pl.ds(c * N_PAD + s * RPS + ch * HC, HC)
        return pltpu.make_async_copy(sets[st][1], out_ref.at[off], semW)

    def _mk_ep_body(st, with_hs):
        pa, pb, h0b, dxb = sets[st]

        def ep_body(i2, carry):
            for r in range(2):
                i = 2 * i2 + r
                for u in range(DH // L):
                    csl = pl.ds(u * L, L)
                    q = pb[i, csl]
                    if with_hs:
                        q = q + h0b[i, csl]
                    pb[i, csl] = dxb[i, csl] * q
            return carry

        return ep_body

    for d in _eload_descs(0, 0):
        d.start()
    for ch in range(4):
        st = ch % 2
        if ch + 1 < 4:
            if ch >= 1:
                _ewdesc(ch - 1, 1 - st).wait()
            for d in _eload_descs(ch + 1, 1 - st):
                d.start()
        for d in _eload_descs(ch, st):
            d.wait()

        @pl.when(c == 0)
        def _():
            lax.fori_loop(0, HC // 2, _mk_ep_body(st, True), 0)

        @pl.when(c != 0)
        def _():
            lax.fori_loop(0, HC // 2, _mk_ep_body(st, False), 0)

        _ewdesc(ch, st).start()
    _ewdesc(2, 0).wait()
    _ewdesc(3, 1).wait()


def _tc_mm_body(x_ref, w1_ref, h0_ref):
    h0_ref[...] = jnp.dot(x_ref[...], w1_ref[...],
                          preferred_element_type=jnp.float32)


def _tc_scale_body(parts_ref, h0_ref, h0s_ref, disx_ref):
    deg = jnp.sum(parts_ref[...], axis=0) + 1.0
    disx = jnp.broadcast_to(lax.rsqrt(deg)[:, None], (N_PAD, DH))
    h0p = jnp.concatenate(
        [h0_ref[...], jnp.zeros((N_PAD - N, DH), jnp.float32)], axis=0)
    h0s_ref[...] = disx * h0p
    disx_ref[...] = disx


def _tc_final_body(r_ref, w2_ref, b2_ref, out_ref):
    rall = r_ref[...]
    agg = rall[0:N] + rall[N_PAD:N_PAD + N]
    out = jnp.dot(agg, w2_ref[...], preferred_element_type=jnp.float32)
    out_ref[...] = out + b2_ref[...]


_tc_mm = pl.pallas_call(
    _tc_mm_body,
    out_shape=jax.ShapeDtypeStruct((N, DH), jnp.float32),
)

_tc_scale = pl.pallas_call(
    _tc_scale_body,
    out_shape=(
        jax.ShapeDtypeStruct((N_PAD, DH), jnp.float32),
        jax.ShapeDtypeStruct((N_PAD, DH), jnp.float32),
    ),
)

_tc_final = pl.pallas_call(
    _tc_final_body,
    out_shape=jax.ShapeDtypeStruct((N, D_OUT), jnp.float32),
)


def kernel(x, edge_index, W1, b1, W2, b2):
    edge3 = edge_index.reshape(2, NW, NBLK, K)
    zeros2 = jnp.zeros((N_PAD, DH), jnp.float32)

    parts = _sc_degree(edge3)
    h0 = _tc_mm(x, W1)
    h0s, disx = _tc_scale(parts, h0)
    p = _sc_scatter1(h0s, edge3, zeros2)
    r = _sc_scatter2(p, h0s, disx, b1, edge3, zeros2)
    out = _tc_final(r, W2, b2.reshape(1, D_OUT))
    return out
